# Initial kernel scaffold; baseline (speedup 1.0000x reference)
import functools
import jax
import jax.numpy as jnp
from jax import lax
from jax.experimental import pallas as pl
from jax.experimental.pallas import tpu as pltpu
from jax.experimental.pallas import tpu_sc as plsc

N = 10000
E = 320000
D = 128
H = 128
C = 40
G = 64

NC = 2
NS = 16
NW = NC * NS
EW = 128
NROWS = E // EW
RPS = N // NS
ZCH = 125

RB = 2000
GRID = N // RB

_mesh = plsc.VectorSubcoreMesh(core_axis_name="c", subcore_axis_name="s")


@functools.partial(
    pl.kernel,
    out_type=jax.ShapeDtypeStruct((NC, N, 16), jnp.float32),
    mesh=_mesh,
    scratch_types=[
        pltpu.VMEM((1, EW), jnp.int32),
        pltpu.VMEM((EW, 16), jnp.float32),
        pltpu.VMEM((ZCH, 16), jnp.float32),
        pltpu.VMEM_SHARED((N, 16), jnp.float32),
    ],
)
def _sc_degree(dst_hbm, out_hbm, didx, ones_v, zbuf, dacc):
    cid = lax.axis_index("c")
    sid = lax.axis_index("s")
    wid = sid * NC + cid

    @pl.loop(0, EW)
    def _(r):
        ones_v[r, pl.ds(0, 16)] = jnp.full((16,), 1.0, jnp.float32)

    @pl.loop(0, ZCH)
    def _(r):
        zbuf[r, pl.ds(0, 16)] = jnp.zeros((16,), jnp.float32)

    @pl.loop(0, RPS, step=ZCH)
    def _(j):
        pltpu.sync_copy(zbuf, dacc.at[pl.ds(sid * RPS + j, ZCH)])

    plsc.subcore_barrier()

    @pl.loop(wid, NROWS, step=NW)
    def _(r):
        pltpu.sync_copy(dst_hbm.at[r], didx.at[0])
        pltpu.sync_copy(ones_v, dacc.at[didx.at[0]], add=True)

    plsc.subcore_barrier()

    @pl.loop(0, RPS, step=ZCH)
    def _(j):
        base = sid * RPS + j
        pltpu.sync_copy(dacc.at[pl.ds(base, ZCH)],
                        out_hbm.at[cid, pl.ds(base, ZCH)])


@functools.partial(
    pl.kernel,
    out_type=jax.ShapeDtypeStruct((NC, N, H), jnp.float32),
    mesh=_mesh,
    scratch_types=[
        pltpu.VMEM((EW,), jnp.int32),
        pltpu.VMEM((1, EW), jnp.int32),
        pltpu.VMEM((EW, H), jnp.float32),
        pltpu.VMEM_SHARED((N, H), jnp.float32),
        pltpu.SemaphoreType.DMA,
    ],
)
def _sc_scatter(y_hbm, src_hbm, dst_hbm, out_hbm, sidx, didx, rows, acc, sem):
    cid = lax.axis_index("c")
    sid = lax.axis_index("s")
    wid = sid * NC + cid

    @pl.loop(0, ZCH)
    def _(r):
        @pl.loop(0, H, step=16)
        def _(col):
            rows[r, pl.ds(col, 16)] = jnp.zeros((16,), jnp.float32)

    @pl.loop(0, RPS, step=ZCH)
    def _(j):
        pltpu.sync_copy(rows.at[pl.ds(0, ZCH)],
                        acc.at[pl.ds(sid * RPS + j, ZCH)])

    plsc.subcore_barrier()

    @pl.loop(wid, NROWS, step=NW)
    def _(r):
        pltpu.sync_copy(src_hbm.at[r], sidx)
        pltpu.sync_copy(dst_hbm.at[r], didx.at[0])
        pltpu.async_copy(y_hbm.at[sidx], rows, sem).wait()
        pltpu.sync_copy(rows, acc.at[didx.at[0]], add=True)

    plsc.subcore_barrier()

    @pl.loop(0, RPS, step=ZCH)
    def _(j):
        base = sid * RPS + j
        pltpu.sync_copy(acc.at[pl.ds(base, ZCH)],
                        out_hbm.at[cid, pl.ds(base, ZCH)])


def _mm_body(x_ref, w_ref, o_ref):
    o_ref[...] = jnp.dot(x_ref[...], w_ref[...],
                         preferred_element_type=jnp.float32)


def _tc_matmul(x, W):
    return pl.pallas_call(
        _mm_body,
        grid=(GRID,),
        in_specs=[pl.BlockSpec((RB, D), lambda i: (i, 0)),
                  pl.BlockSpec((D, H), lambda i: (0, 0))],
        out_specs=pl.BlockSpec((RB, H), lambda i: (i, 0)),
        out_shape=jax.ShapeDtypeStruct((N, H), jnp.float32),
    )(x, W)


def _prep_body(degt_ref, xw_ref, dis_ref, y_ref):
    deg = jnp.sum(degt_ref[0] + degt_ref[1], axis=-1, keepdims=True)
    deg = deg * (1.0 / 16.0) + 1.0
    dis = lax.rsqrt(deg)
    dis_ref[...] = dis
    y_ref[...] = xw_ref[...] * dis


def _tc_prep(degt, xw):
    return pl.pallas_call(
        _prep_body,
        grid=(GRID,),
        in_specs=[pl.BlockSpec((NC, RB, 16), lambda i: (0, i, 0)),
                  pl.BlockSpec((RB, H), lambda i: (i, 0))],
        out_specs=[pl.BlockSpec((RB, 1), lambda i: (i, 0)),
                   pl.BlockSpec((RB, H), lambda i: (i, 0))],
        out_shape=[jax.ShapeDtypeStruct((N, 1), jnp.float32),
                   jax.ShapeDtypeStruct((N, H), jnp.float32)],
    )(degt, xw)


def _ln_relu(t, g, be):
    m = jnp.mean(t, axis=-1, keepdims=True)
    cen = t - m
    v = jnp.mean(cen * cen, axis=-1, keepdims=True)
    h = cen * lax.rsqrt(v + 1e-5) * g + be
    return jnp.maximum(h, 0.0)


def _layer_body(acc_ref, y_ref, dis_ref, b_ref, g_ref, be_ref, w2_ref, y2_ref):
    dis = dis_ref[...]
    t = (acc_ref[0] + acc_ref[1] + y_ref[...]) * dis + b_ref[...]
    h = _ln_relu(t, g_ref[...], be_ref[...])
    y2_ref[...] = jnp.dot(h, w2_ref[...],
                          preferred_element_type=jnp.float32) * dis


def _tc_layer(acc, y, dis, b, g, be, W2):
    return pl.pallas_call(
        _layer_body,
        grid=(GRID,),
        in_specs=[pl.BlockSpec((NC, RB, H), lambda i: (0, i, 0)),
                  pl.BlockSpec((RB, H), lambda i: (i, 0)),
                  pl.BlockSpec((RB, 1), lambda i: (i, 0)),
                  pl.BlockSpec((1, H), lambda i: (0, 0)),
                  pl.BlockSpec((1, H), lambda i: (0, 0)),
                  pl.BlockSpec((1, H), lambda i: (0, 0)),
                  pl.BlockSpec((H, H), lambda i: (0, 0))],
        out_specs=pl.BlockSpec((RB, H), lambda i: (i, 0)),
        out_shape=jax.ShapeDtypeStruct((N, H), jnp.float32),
    )(acc, y, dis, b, g, be, W2)


def _final_body(acc_ref, y_ref, dis_ref, b_ref, g_ref, be_ref, bf_ref,
                wl_ref, bl_ref, o_ref, psum, pcnt):
    i = pl.program_id(0)

    @pl.when(i == 0)
    def _():
        psum[...] = jnp.zeros_like(psum)
        pcnt[...] = jnp.zeros_like(pcnt)

    dis = dis_ref[...]
    t = (acc_ref[0] + acc_ref[1] + y_ref[...]) * dis + b_ref[...]
    h = _ln_relu(t, g_ref[...], be_ref[...])

    gids = lax.broadcasted_iota(jnp.float32, (RB, G), 1)
    onehot = (bf_ref[...] == gids).astype(jnp.float32)
    psum[...] += lax.dot_general(onehot, h, (((0,), (0,)), ((), ())),
                                 preferred_element_type=jnp.float32)
    pcnt[...] += lax.dot_general(onehot, jnp.ones((RB, H), jnp.float32),
                                 (((0,), (0,)), ((), ())),
                                 preferred_element_type=jnp.float32)

    @pl.when(i == GRID - 1)
    def _():
        pooled = psum[...] / jnp.maximum(pcnt[...], 1.0)
        o_ref[...] = jnp.dot(pooled, wl_ref[...],
                             preferred_element_type=jnp.float32) + bl_ref[...]


def _tc_final(acc, y, dis, b, g, be, bf, Wl, bl):
    return pl.pallas_call(
        _final_body,
        grid=(GRID,),
        in_specs=[pl.BlockSpec((NC, RB, H), lambda i: (0, i, 0)),
                  pl.BlockSpec((RB, H), lambda i: (i, 0)),
                  pl.BlockSpec((RB, 1), lambda i: (i, 0)),
                  pl.BlockSpec((1, H), lambda i: (0, 0)),
                  pl.BlockSpec((1, H), lambda i: (0, 0)),
                  pl.BlockSpec((1, H), lambda i: (0, 0)),
                  pl.BlockSpec((RB, 1), lambda i: (i, 0)),
                  pl.BlockSpec((H, C), lambda i: (0, 0)),
                  pl.BlockSpec((1, C), lambda i: (0, 0))],
        out_specs=pl.BlockSpec((G, C), lambda i: (0, 0)),
        out_shape=jax.ShapeDtypeStruct((G, C), jnp.float32),
        scratch_shapes=[pltpu.VMEM((G, H), jnp.float32),
                        pltpu.VMEM((G, H), jnp.float32)],
    )(acc, y, dis, b, g, be, bf, Wl, bl)


def kernel(x, edge_index, batch, W1, b1, g1, be1, W2, b2, g2, be2, Wl, bl):
    src = edge_index[0].reshape(NROWS, EW)
    dst = edge_index[1].reshape(NROWS, EW)
    bf = batch.astype(jnp.float32).reshape(N, 1)
    b1r, g1r, be1r = b1.reshape(1, H), g1.reshape(1, H), be1.reshape(1, H)
    b2r, g2r, be2r = b2.reshape(1, H), g2.reshape(1, H), be2.reshape(1, H)
    blr = bl.reshape(1, C)

    degt = _sc_degree(dst)
    xw1 = _tc_matmul(x, W1)
    dis, y1 = _tc_prep(degt, xw1)
    acc1 = _sc_scatter(y1, src, dst)
    y2 = _tc_layer(acc1, y1, dis, b1r, g1r, be1r, W2)
    acc2 = _sc_scatter(y2, src, dst)
    return _tc_final(acc2, y2, dis, b2r, g2r, be2r, bf, Wl, blr)


# speedup vs baseline: 16.7397x; 16.7397x over previous
import functools
import jax
import jax.numpy as jnp
from jax import lax
from jax.experimental import pallas as pl
from jax.experimental.pallas import tpu as pltpu
from jax.experimental.pallas import tpu_sc as plsc

N = 10000
E = 320000
D = 128
H = 128
C = 40
G = 64

NC = 2
NS = 16
NW = NC * NS
EW = 128
NROWS = E // EW
ZCH = 80
NZ = N // ZCH

RB = 2000
GRID = N // RB


@functools.cache
def _sc_kernels():
    mesh = plsc.VectorSubcoreMesh(core_axis_name="c", subcore_axis_name="s")

    deg_kernel = functools.partial(
        pl.kernel,
        out_type=jax.ShapeDtypeStruct((NC, N, 16), jnp.float32),
        mesh=mesh,
        scratch_types=[
            pltpu.VMEM((1, EW), jnp.int32),
            pltpu.VMEM((EW, 16), jnp.float32),
            pltpu.VMEM((ZCH, 16), jnp.float32),
            pltpu.VMEM_SHARED((N, 16), jnp.float32),
        ],
    )(_sc_degree_body)

    scat_kernel = functools.partial(
        pl.kernel,
        out_type=jax.ShapeDtypeStruct((NC, N, H), jnp.float32),
        mesh=mesh,
        scratch_types=[
            pltpu.VMEM((1, EW), jnp.int32),
            pltpu.VMEM((1, EW), jnp.int32),
            pltpu.VMEM((EW, H), jnp.float32),
            pltpu.VMEM_SHARED((N, H), jnp.float32),
            pltpu.SemaphoreType.DMA,
        ],
    )(_sc_scatter_body)

    return deg_kernel, scat_kernel


def _sc_degree_body(dst_hbm, out_hbm, didx, ones_v, zbuf, dacc):
    cid = lax.axis_index("c")
    sid = lax.axis_index("s")
    wid = sid * NC + cid

    @pl.loop(0, EW)
    def _(r):
        ones_v[r, pl.ds(0, 16)] = jnp.full((16,), 1.0, jnp.float32)

    @pl.loop(0, ZCH)
    def _(r):
        zbuf[r, pl.ds(0, 16)] = jnp.zeros((16,), jnp.float32)

    @pl.loop(sid, NZ, step=NS)
    def _(k):
        pltpu.sync_copy(zbuf, dacc.at[pl.ds(k * ZCH, ZCH)])

    plsc.subcore_barrier()

    @pl.loop(wid, NROWS, step=NW)
    def _(r):
        pltpu.sync_copy(dst_hbm.at[r], didx)
        pltpu.sync_copy(ones_v, dacc.at[didx.at[0]], add=True)

    plsc.subcore_barrier()

    @pl.loop(sid, NZ, step=NS)
    def _(k):
        pltpu.sync_copy(dacc.at[pl.ds(k * ZCH, ZCH)],
                        out_hbm.at[cid, pl.ds(k * ZCH, ZCH)])


def _sc_scatter_body(y_hbm, src_hbm, dst_hbm, out_hbm, sidx, didx, rows, acc, sem):
    cid = lax.axis_index("c")
    sid = lax.axis_index("s")
    wid = sid * NC + cid

    @pl.loop(0, ZCH)
    def _(r):
        @pl.loop(0, H, step=16)
        def _(col):
            rows[r, pl.ds(col, 16)] = jnp.zeros((16,), jnp.float32)

    @pl.loop(sid, NZ, step=NS)
    def _(k):
        pltpu.sync_copy(rows.at[pl.ds(0, ZCH)],
                        acc.at[pl.ds(k * ZCH, ZCH)])

    plsc.subcore_barrier()

    @pl.loop(wid, NROWS, step=NW)
    def _(r):
        pltpu.sync_copy(src_hbm.at[r], sidx)
        pltpu.sync_copy(dst_hbm.at[r], didx)
        pltpu.async_copy(y_hbm.at[sidx.at[0]], rows, sem).wait()
        pltpu.sync_copy(rows, acc.at[didx.at[0]], add=True)

    plsc.subcore_barrier()

    @pl.loop(sid, NZ, step=NS)
    def _(k):
        pltpu.sync_copy(acc.at[pl.ds(k * ZCH, ZCH)],
                        out_hbm.at[cid, pl.ds(k * ZCH, ZCH)])


def _mm_body(x_ref, w_ref, o_ref):
    o_ref[...] = jnp.dot(x_ref[...], w_ref[...],
                         preferred_element_type=jnp.float32)


def _tc_matmul(x, W):
    return pl.pallas_call(
        _mm_body,
        grid=(GRID,),
        in_specs=[pl.BlockSpec((RB, D), lambda i: (i, 0)),
                  pl.BlockSpec((D, H), lambda i: (0, 0))],
        out_specs=pl.BlockSpec((RB, H), lambda i: (i, 0)),
        out_shape=jax.ShapeDtypeStruct((N, H), jnp.float32),
    )(x, W)


def _prep_body(degt_ref, xw_ref, dis_ref, y_ref):
    deg = jnp.sum(degt_ref[0] + degt_ref[1], axis=-1, keepdims=True)
    deg = deg * (1.0 / 16.0) + 1.0
    dis = lax.rsqrt(deg)
    dis_ref[...] = dis
    y_ref[...] = xw_ref[...] * dis


def _tc_prep(degt, xw):
    return pl.pallas_call(
        _prep_body,
        grid=(GRID,),
        in_specs=[pl.BlockSpec((NC, RB, 16), lambda i: (0, i, 0)),
                  pl.BlockSpec((RB, H), lambda i: (i, 0))],
        out_specs=[pl.BlockSpec((RB, 1), lambda i: (i, 0)),
                   pl.BlockSpec((RB, H), lambda i: (i, 0))],
        out_shape=[jax.ShapeDtypeStruct((N, 1), jnp.float32),
                   jax.ShapeDtypeStruct((N, H), jnp.float32)],
    )(degt, xw)


def _ln_relu(t, g, be):
    m = jnp.mean(t, axis=-1, keepdims=True)
    cen = t - m
    v = jnp.mean(cen * cen, axis=-1, keepdims=True)
    h = cen * lax.rsqrt(v + 1e-5) * g + be
    return jnp.maximum(h, 0.0)


def _layer_body(acc_ref, y_ref, dis_ref, b_ref, g_ref, be_ref, w2_ref, y2_ref):
    dis = dis_ref[...]
    t = (acc_ref[0] + acc_ref[1] + y_ref[...]) * dis + b_ref[...]
    h = _ln_relu(t, g_ref[...], be_ref[...])
    y2_ref[...] = jnp.dot(h, w2_ref[...],
                          preferred_element_type=jnp.float32) * dis


def _tc_layer(acc, y, dis, b, g, be, W2):
    return pl.pallas_call(
        _layer_body,
        grid=(GRID,),
        in_specs=[pl.BlockSpec((NC, RB, H), lambda i: (0, i, 0)),
                  pl.BlockSpec((RB, H), lambda i: (i, 0)),
                  pl.BlockSpec((RB, 1), lambda i: (i, 0)),
                  pl.BlockSpec((1, H), lambda i: (0, 0)),
                  pl.BlockSpec((1, H), lambda i: (0, 0)),
                  pl.BlockSpec((1, H), lambda i: (0, 0)),
                  pl.BlockSpec((H, H), lambda i: (0, 0))],
        out_specs=pl.BlockSpec((RB, H), lambda i: (i, 0)),
        out_shape=jax.ShapeDtypeStruct((N, H), jnp.float32),
    )(acc, y, dis, b, g, be, W2)


def _final_body(acc_ref, y_ref, dis_ref, b_ref, g_ref, be_ref, bf_ref,
                wl_ref, bl_ref, o_ref, psum, pcnt):
    i = pl.program_id(0)

    @pl.when(i == 0)
    def _():
        psum[...] = jnp.zeros_like(psum)
        pcnt[...] = jnp.zeros_like(pcnt)

    dis = dis_ref[...]
    t = (acc_ref[0] + acc_ref[1] + y_ref[...]) * dis + b_ref[...]
    h = _ln_relu(t, g_ref[...], be_ref[...])

    gids = lax.broadcasted_iota(jnp.int32, (RB, G), 1)
    onehot = (bf_ref[...] == gids).astype(jnp.float32)
    psum[...] += lax.dot_general(onehot, h, (((0,), (0,)), ((), ())),
                                 preferred_element_type=jnp.float32)
    pcnt[...] += lax.dot_general(onehot, jnp.ones((RB, H), jnp.float32),
                                 (((0,), (0,)), ((), ())),
                                 preferred_element_type=jnp.float32)

    @pl.when(i == GRID - 1)
    def _():
        pooled = psum[...] / jnp.maximum(pcnt[...], 1.0)
        o_ref[...] = jnp.dot(pooled, wl_ref[...],
                             preferred_element_type=jnp.float32) + bl_ref[...]


def _tc_final(acc, y, dis, b, g, be, bf, Wl, bl):
    return pl.pallas_call(
        _final_body,
        grid=(GRID,),
        in_specs=[pl.BlockSpec((NC, RB, H), lambda i: (0, i, 0)),
                  pl.BlockSpec((RB, H), lambda i: (i, 0)),
                  pl.BlockSpec((RB, 1), lambda i: (i, 0)),
                  pl.BlockSpec((1, H), lambda i: (0, 0)),
                  pl.BlockSpec((1, H), lambda i: (0, 0)),
                  pl.BlockSpec((1, H), lambda i: (0, 0)),
                  pl.BlockSpec((RB, 1), lambda i: (i, 0)),
                  pl.BlockSpec((H, C), lambda i: (0, 0)),
                  pl.BlockSpec((1, C), lambda i: (0, 0))],
        out_specs=pl.BlockSpec((G, C), lambda i: (0, 0)),
        out_shape=jax.ShapeDtypeStruct((G, C), jnp.float32),
        scratch_shapes=[pltpu.VMEM((G, H), jnp.float32),
                        pltpu.VMEM((G, H), jnp.float32)],
    )(acc, y, dis, b, g, be, bf, Wl, bl)


def kernel(x, edge_index, batch, W1, b1, g1, be1, W2, b2, g2, be2, Wl, bl):
    src = edge_index[0].reshape(NROWS, 1, EW)
    dst = edge_index[1].reshape(NROWS, 1, EW)
    bf = batch.reshape(N, 1)
    b1r, g1r, be1r = b1.reshape(1, H), g1.reshape(1, H), be1.reshape(1, H)
    b2r, g2r, be2r = b2.reshape(1, H), g2.reshape(1, H), be2.reshape(1, H)
    blr = bl.reshape(1, C)

    sc_degree, sc_scatter = _sc_kernels()

    degt = sc_degree(dst)
    xw1 = _tc_matmul(x, W1)
    dis, y1 = _tc_prep(degt, xw1)
    acc1 = sc_scatter(y1, src, dst)
    y2 = _tc_layer(acc1, y1, dis, b1r, g1r, be1r, W2)
    acc2 = sc_scatter(y2, src, dst)
    return _tc_final(acc2, y2, dis, b2r, g2r, be2r, bf, Wl, blr)


# speedup vs baseline: 21.4795x; 1.2831x over previous
import functools
import jax
import jax.numpy as jnp
from jax import lax
from jax.experimental import pallas as pl
from jax.experimental.pallas import tpu as pltpu
from jax.experimental.pallas import tpu_sc as plsc

N = 10000
E = 320000
D = 128
H = 128
C = 40
G = 64

NC = 2
NS = 16
NW = NC * NS
EW = 80
NROWS = E // EW
CPW = NROWS // NW
EPW = E // NW
ZCH = 80
NZ = N // ZCH

RB = 2000
GRID = N // RB


@functools.cache
def _sc_kernels():
    mesh = plsc.VectorSubcoreMesh(core_axis_name="c", subcore_axis_name="s")

    deg_kernel = functools.partial(
        pl.kernel,
        out_type=jax.ShapeDtypeStruct((NC, N, 16), jnp.float32),
        mesh=mesh,
        scratch_types=[
            pltpu.VMEM((1, EW), jnp.int32),
            pltpu.VMEM((EW, 16), jnp.float32),
            pltpu.VMEM((ZCH, 16), jnp.float32),
            pltpu.VMEM_SHARED((N, 16), jnp.float32),
        ],
    )(_sc_degree_body)

    scat_kernel = functools.partial(
        pl.kernel,
        out_type=jax.ShapeDtypeStruct((NC, N, H), jnp.float32),
        mesh=mesh,
        scratch_types=[
            pltpu.VMEM((EPW,), jnp.int32),
            pltpu.VMEM((CPW, 1, EW), jnp.int32),
            pltpu.VMEM((EW, H), jnp.float32),
            pltpu.VMEM((EW, H), jnp.float32),
            pltpu.VMEM_SHARED((N, H), jnp.float32),
            pltpu.SemaphoreType.DMA,
            pltpu.SemaphoreType.DMA,
        ],
    )(_sc_scatter_body)

    return deg_kernel, scat_kernel


def _sc_degree_body(dst_hbm, out_hbm, didx, ones_v, zbuf, dacc):
    cid = lax.axis_index("c")
    sid = lax.axis_index("s")
    wid = sid * NC + cid

    @pl.loop(0, EW)
    def _(r):
        ones_v[r, pl.ds(0, 16)] = jnp.full((16,), 1.0, jnp.float32)

    @pl.loop(0, ZCH)
    def _(r):
        zbuf[r, pl.ds(0, 16)] = jnp.zeros((16,), jnp.float32)

    @pl.loop(sid, NZ, step=NS)
    def _(k):
        pltpu.sync_copy(zbuf, dacc.at[pl.ds(k * ZCH, ZCH)])

    plsc.subcore_barrier()

    @pl.loop(wid, NROWS, step=NW)
    def _(r):
        pltpu.sync_copy(dst_hbm.at[r], didx)
        pltpu.sync_copy(ones_v, dacc.at[didx.at[0]], add=True)

    plsc.subcore_barrier()

    @pl.loop(sid, NZ, step=NS)
    def _(k):
        pltpu.sync_copy(dacc.at[pl.ds(k * ZCH, ZCH)],
                        out_hbm.at[cid, pl.ds(k * ZCH, ZCH)])


def _sc_scatter_body(y_hbm, src_hbm, dst_hbm, out_hbm, sidxb, didxb,
                     rows0, rows1, acc, sem0, sem1):
    cid = lax.axis_index("c")
    sid = lax.axis_index("s")
    wid = sid * NC + cid

    pltpu.sync_copy(src_hbm.at[pl.ds(wid * EPW, EPW)], sidxb)
    pltpu.sync_copy(dst_hbm.at[pl.ds(wid * CPW, CPW)], didxb)

    @pl.loop(0, ZCH)
    def _(r):
        @pl.loop(0, H, step=16)
        def _(col):
            rows0[r, pl.ds(col, 16)] = jnp.zeros((16,), jnp.float32)

    @pl.loop(sid, NZ, step=NS)
    def _(k):
        pltpu.sync_copy(rows0.at[pl.ds(0, ZCH)],
                        acc.at[pl.ds(k * ZCH, ZCH)])

    plsc.subcore_barrier()

    def gather_start(k, rows, sem):
        pltpu.async_copy(y_hbm.at[sidxb.at[pl.ds(k * EW, EW)]], rows, sem)

    def gather_wait(k, rows, sem):
        pltpu.make_async_copy(y_hbm.at[sidxb.at[pl.ds(k * EW, EW)]],
                              rows, sem).wait()

    def scatter_add(k, rows):
        pltpu.sync_copy(rows, acc.at[didxb.at[k, 0]], add=True)

    @pl.loop(0, CPW - 1, step=2)
    def _(k):
        g0 = pltpu.async_copy(y_hbm.at[sidxb.at[pl.ds(k * EW, EW)]],
                              rows0, sem0)
        g1 = pltpu.async_copy(y_hbm.at[sidxb.at[pl.ds((k + 1) * EW, EW)]],
                              rows1, sem1)
        g0.wait()
        scatter_add(k, rows0)
        g1.wait()
        scatter_add(k + 1, rows1)

    gather_start(CPW - 1, rows0, sem0)
    gather_wait(CPW - 1, rows0, sem0)
    scatter_add(CPW - 1, rows0)

    plsc.subcore_barrier()

    @pl.loop(sid, NZ, step=NS)
    def _(k):
        pltpu.sync_copy(acc.at[pl.ds(k * ZCH, ZCH)],
                        out_hbm.at[cid, pl.ds(k * ZCH, ZCH)])


def _mm_body(x_ref, w_ref, o_ref):
    o_ref[...] = jnp.dot(x_ref[...], w_ref[...],
                         preferred_element_type=jnp.float32)


def _tc_matmul(x, W):
    return pl.pallas_call(
        _mm_body,
        grid=(GRID,),
        in_specs=[pl.BlockSpec((RB, D), lambda i: (i, 0)),
                  pl.BlockSpec((D, H), lambda i: (0, 0))],
        out_specs=pl.BlockSpec((RB, H), lambda i: (i, 0)),
        out_shape=jax.ShapeDtypeStruct((N, H), jnp.float32),
    )(x, W)


def _prep_body(degt_ref, xw_ref, dis_ref, y_ref):
    deg = jnp.sum(degt_ref[0] + degt_ref[1], axis=-1, keepdims=True)
    deg = deg * (1.0 / 16.0) + 1.0
    dis = lax.rsqrt(deg)
    dis_ref[...] = dis
    y_ref[...] = xw_ref[...] * dis


def _tc_prep(degt, xw):
    return pl.pallas_call(
        _prep_body,
        grid=(GRID,),
        in_specs=[pl.BlockSpec((NC, RB, 16), lambda i: (0, i, 0)),
                  pl.BlockSpec((RB, H), lambda i: (i, 0))],
        out_specs=[pl.BlockSpec((RB, 1), lambda i: (i, 0)),
                   pl.BlockSpec((RB, H), lambda i: (i, 0))],
        out_shape=[jax.ShapeDtypeStruct((N, 1), jnp.float32),
                   jax.ShapeDtypeStruct((N, H), jnp.float32)],
    )(degt, xw)


def _ln_relu(t, g, be):
    m = jnp.mean(t, axis=-1, keepdims=True)
    cen = t - m
    v = jnp.mean(cen * cen, axis=-1, keepdims=True)
    h = cen * lax.rsqrt(v + 1e-5) * g + be
    return jnp.maximum(h, 0.0)


def _layer_body(acc_ref, y_ref, dis_ref, b_ref, g_ref, be_ref, w2_ref, y2_ref):
    dis = dis_ref[...]
    t = (acc_ref[0] + acc_ref[1] + y_ref[...]) * dis + b_ref[...]
    h = _ln_relu(t, g_ref[...], be_ref[...])
    y2_ref[...] = jnp.dot(h, w2_ref[...],
                          preferred_element_type=jnp.float32) * dis


def _tc_layer(acc, y, dis, b, g, be, W2):
    return pl.pallas_call(
        _layer_body,
        grid=(GRID,),
        in_specs=[pl.BlockSpec((NC, RB, H), lambda i: (0, i, 0)),
                  pl.BlockSpec((RB, H), lambda i: (i, 0)),
                  pl.BlockSpec((RB, 1), lambda i: (i, 0)),
                  pl.BlockSpec((1, H), lambda i: (0, 0)),
                  pl.BlockSpec((1, H), lambda i: (0, 0)),
                  pl.BlockSpec((1, H), lambda i: (0, 0)),
                  pl.BlockSpec((H, H), lambda i: (0, 0))],
        out_specs=pl.BlockSpec((RB, H), lambda i: (i, 0)),
        out_shape=jax.ShapeDtypeStruct((N, H), jnp.float32),
    )(acc, y, dis, b, g, be, W2)


def _final_body(acc_ref, y_ref, dis_ref, b_ref, g_ref, be_ref, bf_ref,
                wl_ref, bl_ref, o_ref, psum, pcnt):
    i = pl.program_id(0)

    @pl.when(i == 0)
    def _():
        psum[...] = jnp.zeros_like(psum)
        pcnt[...] = jnp.zeros_like(pcnt)

    dis = dis_ref[...]
    t = (acc_ref[0] + acc_ref[1] + y_ref[...]) * dis + b_ref[...]
    h = _ln_relu(t, g_ref[...], be_ref[...])

    gids = lax.broadcasted_iota(jnp.int32, (RB, G), 1)
    onehot = (bf_ref[...] == gids).astype(jnp.float32)
    psum[...] += lax.dot_general(onehot, h, (((0,), (0,)), ((), ())),
                                 preferred_element_type=jnp.float32)
    pcnt[...] += lax.dot_general(onehot, jnp.ones((RB, H), jnp.float32),
                                 (((0,), (0,)), ((), ())),
                                 preferred_element_type=jnp.float32)

    @pl.when(i == GRID - 1)
    def _():
        pooled = psum[...] / jnp.maximum(pcnt[...], 1.0)
        o_ref[...] = jnp.dot(pooled, wl_ref[...],
                             preferred_element_type=jnp.float32) + bl_ref[...]


def _tc_final(acc, y, dis, b, g, be, bf, Wl, bl):
    return pl.pallas_call(
        _final_body,
        grid=(GRID,),
        in_specs=[pl.BlockSpec((NC, RB, H), lambda i: (0, i, 0)),
                  pl.BlockSpec((RB, H), lambda i: (i, 0)),
                  pl.BlockSpec((RB, 1), lambda i: (i, 0)),
                  pl.BlockSpec((1, H), lambda i: (0, 0)),
                  pl.BlockSpec((1, H), lambda i: (0, 0)),
                  pl.BlockSpec((1, H), lambda i: (0, 0)),
                  pl.BlockSpec((RB, 1), lambda i: (i, 0)),
                  pl.BlockSpec((H, C), lambda i: (0, 0)),
                  pl.BlockSpec((1, C), lambda i: (0, 0))],
        out_specs=pl.BlockSpec((G, C), lambda i: (0, 0)),
        out_shape=jax.ShapeDtypeStruct((G, C), jnp.float32),
        scratch_shapes=[pltpu.VMEM((G, H), jnp.float32),
                        pltpu.VMEM((G, H), jnp.float32)],
    )(acc, y, dis, b, g, be, bf, Wl, bl)


def kernel(x, edge_index, batch, W1, b1, g1, be1, W2, b2, g2, be2, Wl, bl):
    src = edge_index[0]
    dst = edge_index[1].reshape(NROWS, 1, EW)
    bf = batch.reshape(N, 1)
    b1r, g1r, be1r = b1.reshape(1, H), g1.reshape(1, H), be1.reshape(1, H)
    b2r, g2r, be2r = b2.reshape(1, H), g2.reshape(1, H), be2.reshape(1, H)
    blr = bl.reshape(1, C)

    sc_degree, sc_scatter = _sc_kernels()

    degt = sc_degree(dst)
    xw1 = _tc_matmul(x, W1)
    dis, y1 = _tc_prep(degt, xw1)
    acc1 = sc_scatter(y1, src, dst)
    y2 = _tc_layer(acc1, y1, dis, b1r, g1r, be1r, W2)
    acc2 = sc_scatter(y2, src, dst)
    return _tc_final(acc2, y2, dis, b2r, g2r, be2r, bf, Wl, blr)


# speedup vs baseline: 21.7335x; 1.0118x over previous
import functools
import jax
import jax.numpy as jnp
from jax import lax
from jax.experimental import pallas as pl
from jax.experimental.pallas import tpu as pltpu
from jax.experimental.pallas import tpu_sc as plsc

N = 10000
E = 320000
D = 128
H = 128
C = 40
G = 64

NC = 2
NS = 16
NW = NC * NS
EW = 80
NROWS = E // EW
CPW = NROWS // NW
EPW = E // NW
ZCH = 80
NZ = N // ZCH

RB = 2000
GRID = N // RB


@functools.cache
def _sc_kernels():
    mesh = plsc.VectorSubcoreMesh(core_axis_name="c", subcore_axis_name="s")

    deg_kernel = functools.partial(
        pl.kernel,
        out_type=jax.ShapeDtypeStruct((NC, N, 16), jnp.float32),
        mesh=mesh,
        scratch_types=[
            pltpu.VMEM((1, EW), jnp.int32),
            pltpu.VMEM((EW, 16), jnp.float32),
            pltpu.VMEM((ZCH, 16), jnp.float32),
            pltpu.VMEM_SHARED((N, 16), jnp.float32),
        ],
    )(_sc_degree_body)

    scat_kernel = functools.partial(
        pl.kernel,
        out_type=jax.ShapeDtypeStruct((NC, N, H), jnp.float32),
        mesh=mesh,
        scratch_types=[
            pltpu.VMEM((EPW,), jnp.int32),
            pltpu.VMEM((CPW, 1, EW), jnp.int32),
            pltpu.VMEM((EW, H), jnp.float32),
            pltpu.VMEM((EW, H), jnp.float32),
            pltpu.VMEM_SHARED((N, H), jnp.float32),
            pltpu.SemaphoreType.DMA,
            pltpu.SemaphoreType.DMA,
            pltpu.SemaphoreType.DMA,
            pltpu.SemaphoreType.DMA,
        ],
    )(_sc_scatter_body)

    return deg_kernel, scat_kernel


def _sc_degree_body(dst_hbm, out_hbm, didx, ones_v, zbuf, dacc):
    cid = lax.axis_index("c")
    sid = lax.axis_index("s")
    wid = sid * NC + cid

    @pl.loop(0, EW)
    def _(r):
        ones_v[r, pl.ds(0, 16)] = jnp.full((16,), 1.0, jnp.float32)

    @pl.loop(0, ZCH)
    def _(r):
        zbuf[r, pl.ds(0, 16)] = jnp.zeros((16,), jnp.float32)

    @pl.loop(sid, NZ, step=NS)
    def _(k):
        pltpu.sync_copy(zbuf, dacc.at[pl.ds(k * ZCH, ZCH)])

    plsc.subcore_barrier()

    @pl.loop(wid, NROWS, step=NW)
    def _(r):
        pltpu.sync_copy(dst_hbm.at[r], didx)
        pltpu.sync_copy(ones_v, dacc.at[didx.at[0]], add=True)

    plsc.subcore_barrier()

    @pl.loop(sid, NZ, step=NS)
    def _(k):
        pltpu.sync_copy(dacc.at[pl.ds(k * ZCH, ZCH)],
                        out_hbm.at[cid, pl.ds(k * ZCH, ZCH)])


def _sc_scatter_body(y_hbm, src_hbm, dst_hbm, out_hbm, sidxb, didxb,
                     rows0, rows1, acc, sem0, sem1, sems0, sems1):
    cid = lax.axis_index("c")
    sid = lax.axis_index("s")
    wid = sid * NC + cid

    ld0 = pltpu.async_copy(src_hbm.at[pl.ds(wid * EPW, EPW)], sidxb, sem0)
    ld1 = pltpu.async_copy(dst_hbm.at[pl.ds(wid * CPW, CPW)], didxb, sem1)

    @pl.loop(0, ZCH)
    def _(r):
        @pl.loop(0, H, step=16)
        def _(col):
            rows0[r, pl.ds(col, 16)] = jnp.zeros((16,), jnp.float32)

    @pl.loop(sid, NZ, step=NS)
    def _(k):
        pltpu.sync_copy(rows0.at[pl.ds(0, ZCH)],
                        acc.at[pl.ds(k * ZCH, ZCH)])

    ld0.wait()
    ld1.wait()
    plsc.subcore_barrier()

    def gather_start(k, rows, sem):
        pltpu.async_copy(y_hbm.at[sidxb.at[pl.ds(k * EW, EW)]], rows, sem)

    def gather_wait(k, rows, sem):
        pltpu.make_async_copy(y_hbm.at[sidxb.at[pl.ds(k * EW, EW)]],
                              rows, sem).wait()

    def scatter_add(k, rows):
        pltpu.sync_copy(rows, acc.at[didxb.at[k, 0]], add=True)

    @pl.loop(0, CPW - 1, step=2)
    def _(k):
        g0 = pltpu.async_copy(y_hbm.at[sidxb.at[pl.ds(k * EW, EW)]],
                              rows0, sem0)
        g1 = pltpu.async_copy(y_hbm.at[sidxb.at[pl.ds((k + 1) * EW, EW)]],
                              rows1, sem1)
        g0.wait()
        scatter_add(k, rows0)
        g1.wait()
        scatter_add(k + 1, rows1)

    gather_start(CPW - 1, rows0, sem0)
    gather_wait(CPW - 1, rows0, sem0)
    scatter_add(CPW - 1, rows0)

    plsc.subcore_barrier()

    @pl.loop(sid, NZ, step=NS)
    def _(k):
        pltpu.sync_copy(acc.at[pl.ds(k * ZCH, ZCH)],
                        out_hbm.at[cid, pl.ds(k * ZCH, ZCH)])


def _mm_body(x_ref, w_ref, o_ref):
    o_ref[...] = jnp.dot(x_ref[...], w_ref[...],
                         preferred_element_type=jnp.float32)


def _tc_matmul(x, W):
    return pl.pallas_call(
        _mm_body,
        grid=(GRID,),
        in_specs=[pl.BlockSpec((RB, D), lambda i: (i, 0)),
                  pl.BlockSpec((D, H), lambda i: (0, 0))],
        out_specs=pl.BlockSpec((RB, H), lambda i: (i, 0)),
        out_shape=jax.ShapeDtypeStruct((N, H), jnp.float32),
    )(x, W)


def _prep_body(degt_ref, xw_ref, dis_ref, y_ref):
    deg = jnp.sum(degt_ref[0] + degt_ref[1], axis=-1, keepdims=True)
    deg = deg * (1.0 / 16.0) + 1.0
    dis = lax.rsqrt(deg)
    dis_ref[...] = dis
    y_ref[...] = xw_ref[...] * dis


def _tc_prep(degt, xw):
    return pl.pallas_call(
        _prep_body,
        grid=(GRID,),
        in_specs=[pl.BlockSpec((NC, RB, 16), lambda i: (0, i, 0)),
                  pl.BlockSpec((RB, H), lambda i: (i, 0))],
        out_specs=[pl.BlockSpec((RB, 1), lambda i: (i, 0)),
                   pl.BlockSpec((RB, H), lambda i: (i, 0))],
        out_shape=[jax.ShapeDtypeStruct((N, 1), jnp.float32),
                   jax.ShapeDtypeStruct((N, H), jnp.float32)],
    )(degt, xw)


def _ln_relu(t, g, be):
    m = jnp.mean(t, axis=-1, keepdims=True)
    cen = t - m
    v = jnp.mean(cen * cen, axis=-1, keepdims=True)
    h = cen * lax.rsqrt(v + 1e-5) * g + be
    return jnp.maximum(h, 0.0)


def _layer_body(acc_ref, y_ref, dis_ref, b_ref, g_ref, be_ref, w2_ref, y2_ref):
    dis = dis_ref[...]
    t = (acc_ref[0] + acc_ref[1] + y_ref[...]) * dis + b_ref[...]
    h = _ln_relu(t, g_ref[...], be_ref[...])
    y2_ref[...] = jnp.dot(h, w2_ref[...],
                          preferred_element_type=jnp.float32) * dis


def _tc_layer(acc, y, dis, b, g, be, W2):
    return pl.pallas_call(
        _layer_body,
        grid=(GRID,),
        in_specs=[pl.BlockSpec((NC, RB, H), lambda i: (0, i, 0)),
                  pl.BlockSpec((RB, H), lambda i: (i, 0)),
                  pl.BlockSpec((RB, 1), lambda i: (i, 0)),
                  pl.BlockSpec((1, H), lambda i: (0, 0)),
                  pl.BlockSpec((1, H), lambda i: (0, 0)),
                  pl.BlockSpec((1, H), lambda i: (0, 0)),
                  pl.BlockSpec((H, H), lambda i: (0, 0))],
        out_specs=pl.BlockSpec((RB, H), lambda i: (i, 0)),
        out_shape=jax.ShapeDtypeStruct((N, H), jnp.float32),
    )(acc, y, dis, b, g, be, W2)


def _final_body(acc_ref, y_ref, dis_ref, b_ref, g_ref, be_ref, bf_ref,
                wl_ref, bl_ref, o_ref, psum, pcnt):
    i = pl.program_id(0)

    @pl.when(i == 0)
    def _():
        psum[...] = jnp.zeros_like(psum)
        pcnt[...] = jnp.zeros_like(pcnt)

    dis = dis_ref[...]
    t = (acc_ref[0] + acc_ref[1] + y_ref[...]) * dis + b_ref[...]
    h = _ln_relu(t, g_ref[...], be_ref[...])

    gids = lax.broadcasted_iota(jnp.int32, (RB, G), 1)
    onehot = (bf_ref[...] == gids).astype(jnp.float32)
    psum[...] += lax.dot_general(onehot, h, (((0,), (0,)), ((), ())),
                                 preferred_element_type=jnp.float32)
    pcnt[...] += lax.dot_general(onehot, jnp.ones((RB, H), jnp.float32),
                                 (((0,), (0,)), ((), ())),
                                 preferred_element_type=jnp.float32)

    @pl.when(i == GRID - 1)
    def _():
        pooled = psum[...] / jnp.maximum(pcnt[...], 1.0)
        o_ref[...] = jnp.dot(pooled, wl_ref[...],
                             preferred_element_type=jnp.float32) + bl_ref[...]


def _tc_final(acc, y, dis, b, g, be, bf, Wl, bl):
    return pl.pallas_call(
        _final_body,
        grid=(GRID,),
        in_specs=[pl.BlockSpec((NC, RB, H), lambda i: (0, i, 0)),
                  pl.BlockSpec((RB, H), lambda i: (i, 0)),
                  pl.BlockSpec((RB, 1), lambda i: (i, 0)),
                  pl.BlockSpec((1, H), lambda i: (0, 0)),
                  pl.BlockSpec((1, H), lambda i: (0, 0)),
                  pl.BlockSpec((1, H), lambda i: (0, 0)),
                  pl.BlockSpec((RB, 1), lambda i: (i, 0)),
                  pl.BlockSpec((H, C), lambda i: (0, 0)),
                  pl.BlockSpec((1, C), lambda i: (0, 0))],
        out_specs=pl.BlockSpec((G, C), lambda i: (0, 0)),
        out_shape=jax.ShapeDtypeStruct((G, C), jnp.float32),
        scratch_shapes=[pltpu.VMEM((G, H), jnp.float32),
                        pltpu.VMEM((G, H), jnp.float32)],
    )(acc, y, dis, b, g, be, bf, Wl, bl)


def kernel(x, edge_index, batch, W1, b1, g1, be1, W2, b2, g2, be2, Wl, bl):
    src = edge_index[0]
    dst = edge_index[1].reshape(NROWS, 1, EW)
    bf = batch.reshape(N, 1)
    b1r, g1r, be1r = b1.reshape(1, H), g1.reshape(1, H), be1.reshape(1, H)
    b2r, g2r, be2r = b2.reshape(1, H), g2.reshape(1, H), be2.reshape(1, H)
    blr = bl.reshape(1, C)

    sc_degree, sc_scatter = _sc_kernels()

    degt = sc_degree(dst)
    xw1 = _tc_matmul(x, W1)
    dis, y1 = _tc_prep(degt, xw1)
    acc1 = sc_scatter(y1, src, dst)
    y2 = _tc_layer(acc1, y1, dis, b1r, g1r, be1r, W2)
    acc2 = sc_scatter(y2, src, dst)
    return _tc_final(acc2, y2, dis, b2r, g2r, be2r, bf, Wl, blr)


# speedup vs baseline: 24.1016x; 1.1090x over previous
import functools
import jax
import jax.numpy as jnp
from jax import lax
from jax.experimental import pallas as pl
from jax.experimental.pallas import tpu as pltpu
from jax.experimental.pallas import tpu_sc as plsc

N = 10000
E = 320000
D = 128
H = 128
C = 40
G = 64

NC = 2
NS = 16
NW = NC * NS
EW = 128
NROWS = E // EW
CPW = NROWS // NW
NCH = CPW + 1
ZCH = 80
NZ = N // ZCH

RB = 2000
GRID = N // RB


@functools.cache
def _sc_kernels():
    mesh = plsc.VectorSubcoreMesh(core_axis_name="c", subcore_axis_name="s")

    deg_kernel = functools.partial(
        pl.kernel,
        out_type=jax.ShapeDtypeStruct((NC, N, H), jnp.float32),
        mesh=mesh,
        scratch_types=[
            pltpu.VMEM((NCH, 1, EW), jnp.int32),
            pltpu.VMEM((EW, H), jnp.float32),
            pltpu.VMEM((ZCH, H), jnp.float32),
            pltpu.VMEM_SHARED((N, H), jnp.float32),
            pltpu.SemaphoreType.DMA,
        ],
    )(_sc_degree_body)

    scat_kernel = functools.partial(
        pl.kernel,
        out_type=jax.ShapeDtypeStruct((NC, N, H), jnp.float32),
        mesh=mesh,
        scratch_types=[
            pltpu.VMEM((EW,), jnp.int32),
            pltpu.VMEM((EW,), jnp.int32),
            pltpu.VMEM((EW,), jnp.int32),
            pltpu.VMEM((EW,), jnp.int32),
            pltpu.VMEM((NCH, 1, EW), jnp.int32),
            pltpu.VMEM((EW, H), jnp.float32),
            pltpu.VMEM((EW, H), jnp.float32),
            pltpu.VMEM_SHARED((N, H), jnp.float32),
            pltpu.SemaphoreType.DMA,
            pltpu.SemaphoreType.DMA,
            pltpu.SemaphoreType.DMA,
            pltpu.SemaphoreType.DMA,
            pltpu.SemaphoreType.DMA,
            pltpu.SemaphoreType.DMA,
        ],
    )(_sc_scatter_body)

    return deg_kernel, scat_kernel


def _sc_degree_body(dst_hbm, out_hbm, didxb, ones_v, zbuf, dacc, sem0):
    cid = lax.axis_index("c")
    sid = lax.axis_index("s")
    wid = sid * NC + cid

    start = wid * CPW + jnp.minimum(wid, 4)
    ld = pltpu.async_copy(dst_hbm.at[pl.ds(start, CPW)],
                          didxb.at[pl.ds(0, CPW)], sem0)

    @pl.when(wid < 4)
    def _():
        pltpu.sync_copy(dst_hbm.at[start + CPW], didxb.at[CPW])

    @pl.loop(0, EW)
    def _(r):
        @pl.loop(0, H, step=16)
        def _(c):
            ones_v[r, pl.ds(c, 16)] = jnp.full((16,), 1.0, jnp.float32)

    @pl.loop(0, ZCH)
    def _(r):
        @pl.loop(0, H, step=16)
        def _(c):
            zbuf[r, pl.ds(c, 16)] = jnp.zeros((16,), jnp.float32)

    @pl.loop(sid, NZ, step=NS)
    def _(k):
        pltpu.sync_copy(zbuf, dacc.at[pl.ds(k * ZCH, ZCH)])

    ld.wait()
    plsc.subcore_barrier()

    @pl.loop(0, CPW)
    def _(k):
        pltpu.sync_copy(ones_v, dacc.at[didxb.at[k, 0]], add=True)

    @pl.when(wid < 4)
    def _():
        pltpu.sync_copy(ones_v, dacc.at[didxb.at[CPW, 0]], add=True)

    plsc.subcore_barrier()

    @pl.loop(sid, NZ, step=NS)
    def _(k):
        pltpu.sync_copy(dacc.at[pl.ds(k * ZCH, ZCH)],
                        out_hbm.at[cid, pl.ds(k * ZCH, ZCH)])


def _sc_scatter_body(y_hbm, src_hbm, dst_hbm, out_hbm, s0, s1, s2, s3,
                     didxb, rows0, rows1, acc,
                     semi0, semi1, semi2, semi3, semg0, semg1):
    cid = lax.axis_index("c")
    sid = lax.axis_index("s")
    wid = sid * NC + cid

    start = wid * CPW + jnp.minimum(wid, 4)
    nch = CPW + jnp.where(wid < 4, 1, 0)

    ldd = pltpu.async_copy(dst_hbm.at[pl.ds(start, CPW)],
                           didxb.at[pl.ds(0, CPW)], semg0)

    @pl.when(wid < 4)
    def _():
        pltpu.sync_copy(dst_hbm.at[start + CPW], didxb.at[CPW])

    @pl.loop(0, ZCH)
    def _(r):
        @pl.loop(0, H, step=16)
        def _(col):
            rows0[r, pl.ds(col, 16)] = jnp.zeros((16,), jnp.float32)

    @pl.loop(sid, NZ, step=NS)
    def _(k):
        pltpu.sync_copy(rows0.at[pl.ds(0, ZCH)],
                        acc.at[pl.ds(k * ZCH, ZCH)])

    ldd.wait()
    plsc.subcore_barrier()

    def idx_load(j, sbuf, sem):
        jc = jnp.minimum(j, nch - 1)
        return pltpu.async_copy(src_hbm.at[pl.ds((start + jc) * EW, EW)],
                                sbuf, sem)

    def gather(sbuf, rows, sem):
        return pltpu.async_copy(y_hbm.at[sbuf], rows, sem)

    def scatter_add(j, rows):
        pltpu.sync_copy(rows, acc.at[didxb.at[j, 0]], add=True)

    l0 = idx_load(0, s0, semi0)
    l1 = idx_load(1, s1, semi1)
    l2 = idx_load(2, s2, semi2)
    l3 = idx_load(3, s3, semi3)
    l0.wait()
    l1.wait()
    l2.wait()
    l3.wait()

    @pl.loop(0, CPW - 2, step=4)
    def _(j):
        g0 = gather(s0, rows0, semg0)
        g1 = gather(s1, rows1, semg1)
        g0.wait()
        n0 = idx_load(j + 4, s0, semi0)
        scatter_add(j, rows0)
        g1.wait()
        n1 = idx_load(j + 5, s1, semi1)
        g2 = gather(s2, rows0, semg0)
        scatter_add(j + 1, rows1)
        g2.wait()
        n2 = idx_load(j + 6, s2, semi2)
        g3 = gather(s3, rows1, semg1)
        scatter_add(j + 2, rows0)
        g3.wait()
        n3 = idx_load(j + 7, s3, semi3)
        scatter_add(j + 3, rows1)
        n0.wait()
        n1.wait()
        n2.wait()
        n3.wait()

    tg0 = gather(s0, rows0, semg0)
    tg1 = gather(s1, rows1, semg1)
    tg0.wait()
    scatter_add(CPW - 2, rows0)
    tg1.wait()
    scatter_add(CPW - 1, rows1)

    @pl.when(wid < 4)
    def _():
        gather(s2, rows0, semg0).wait()
        scatter_add(CPW, rows0)

    plsc.subcore_barrier()

    @pl.loop(sid, NZ, step=NS)
    def _(k):
        pltpu.sync_copy(acc.at[pl.ds(k * ZCH, ZCH)],
                        out_hbm.at[cid, pl.ds(k * ZCH, ZCH)])


def _mm_body(x_ref, w_ref, o_ref):
    o_ref[...] = jnp.dot(x_ref[...], w_ref[...],
                         preferred_element_type=jnp.float32)


def _tc_matmul(x, W):
    return pl.pallas_call(
        _mm_body,
        grid=(GRID,),
        in_specs=[pl.BlockSpec((RB, D), lambda i: (i, 0)),
                  pl.BlockSpec((D, H), lambda i: (0, 0))],
        out_specs=pl.BlockSpec((RB, H), lambda i: (i, 0)),
        out_shape=jax.ShapeDtypeStruct((N, H), jnp.float32),
    )(x, W)


def _prep_body(degt_ref, xw_ref, dis_ref, y_ref):
    deg = jnp.sum(degt_ref[0] + degt_ref[1], axis=-1, keepdims=True)
    deg = deg * (1.0 / H) + 1.0
    dis = lax.rsqrt(deg)
    dis_ref[...] = dis
    y_ref[...] = xw_ref[...] * dis


def _tc_prep(degt, xw):
    return pl.pallas_call(
        _prep_body,
        grid=(GRID,),
        in_specs=[pl.BlockSpec((NC, RB, H), lambda i: (0, i, 0)),
                  pl.BlockSpec((RB, H), lambda i: (i, 0))],
        out_specs=[pl.BlockSpec((RB, 1), lambda i: (i, 0)),
                   pl.BlockSpec((RB, H), lambda i: (i, 0))],
        out_shape=[jax.ShapeDtypeStruct((N, 1), jnp.float32),
                   jax.ShapeDtypeStruct((N, H), jnp.float32)],
    )(degt, xw)


def _ln_relu(t, g, be):
    m = jnp.mean(t, axis=-1, keepdims=True)
    cen = t - m
    v = jnp.mean(cen * cen, axis=-1, keepdims=True)
    h = cen * lax.rsqrt(v + 1e-5) * g + be
    return jnp.maximum(h, 0.0)


def _layer_body(acc_ref, y_ref, dis_ref, b_ref, g_ref, be_ref, w2_ref, y2_ref):
    dis = dis_ref[...]
    t = (acc_ref[0] + acc_ref[1] + y_ref[...]) * dis + b_ref[...]
    h = _ln_relu(t, g_ref[...], be_ref[...])
    y2_ref[...] = jnp.dot(h, w2_ref[...],
                          preferred_element_type=jnp.float32) * dis


def _tc_layer(acc, y, dis, b, g, be, W2):
    return pl.pallas_call(
        _layer_body,
        grid=(GRID,),
        in_specs=[pl.BlockSpec((NC, RB, H), lambda i: (0, i, 0)),
                  pl.BlockSpec((RB, H), lambda i: (i, 0)),
                  pl.BlockSpec((RB, 1), lambda i: (i, 0)),
                  pl.BlockSpec((1, H), lambda i: (0, 0)),
                  pl.BlockSpec((1, H), lambda i: (0, 0)),
                  pl.BlockSpec((1, H), lambda i: (0, 0)),
                  pl.BlockSpec((H, H), lambda i: (0, 0))],
        out_specs=pl.BlockSpec((RB, H), lambda i: (i, 0)),
        out_shape=jax.ShapeDtypeStruct((N, H), jnp.float32),
    )(acc, y, dis, b, g, be, W2)


def _final_body(acc_ref, y_ref, dis_ref, b_ref, g_ref, be_ref, bf_ref,
                wl_ref, bl_ref, o_ref, psum, pcnt):
    i = pl.program_id(0)

    @pl.when(i == 0)
    def _():
        psum[...] = jnp.zeros_like(psum)
        pcnt[...] = jnp.zeros_like(pcnt)

    dis = dis_ref[...]
    t = (acc_ref[0] + acc_ref[1] + y_ref[...]) * dis + b_ref[...]
    h = _ln_relu(t, g_ref[...], be_ref[...])

    gids = lax.broadcasted_iota(jnp.int32, (RB, G), 1)
    onehot = (bf_ref[...] == gids).astype(jnp.float32)
    psum[...] += lax.dot_general(onehot, h, (((0,), (0,)), ((), ())),
                                 preferred_element_type=jnp.float32)
    pcnt[...] += lax.dot_general(onehot, jnp.ones((RB, H), jnp.float32),
                                 (((0,), (0,)), ((), ())),
                                 preferred_element_type=jnp.float32)

    @pl.when(i == GRID - 1)
    def _():
        pooled = psum[...] / jnp.maximum(pcnt[...], 1.0)
        o_ref[...] = jnp.dot(pooled, wl_ref[...],
                             preferred_element_type=jnp.float32) + bl_ref[...]


def _tc_final(acc, y, dis, b, g, be, bf, Wl, bl):
    return pl.pallas_call(
        _final_body,
        grid=(GRID,),
        in_specs=[pl.BlockSpec((NC, RB, H), lambda i: (0, i, 0)),
                  pl.BlockSpec((RB, H), lambda i: (i, 0)),
                  pl.BlockSpec((RB, 1), lambda i: (i, 0)),
                  pl.BlockSpec((1, H), lambda i: (0, 0)),
                  pl.BlockSpec((1, H), lambda i: (0, 0)),
                  pl.BlockSpec((1, H), lambda i: (0, 0)),
                  pl.BlockSpec((RB, 1), lambda i: (i, 0)),
                  pl.BlockSpec((H, C), lambda i: (0, 0)),
                  pl.BlockSpec((1, C), lambda i: (0, 0))],
        out_specs=pl.BlockSpec((G, C), lambda i: (0, 0)),
        out_shape=jax.ShapeDtypeStruct((G, C), jnp.float32),
        scratch_shapes=[pltpu.VMEM((G, H), jnp.float32),
                        pltpu.VMEM((G, H), jnp.float32)],
    )(acc, y, dis, b, g, be, bf, Wl, bl)


def kernel(x, edge_index, batch, W1, b1, g1, be1, W2, b2, g2, be2, Wl, bl):
    src = edge_index[0]
    dst = edge_index[1].reshape(NROWS, 1, EW)
    bf = batch.reshape(N, 1)
    b1r, g1r, be1r = b1.reshape(1, H), g1.reshape(1, H), be1.reshape(1, H)
    b2r, g2r, be2r = b2.reshape(1, H), g2.reshape(1, H), be2.reshape(1, H)
    blr = bl.reshape(1, C)

    sc_degree, sc_scatter = _sc_kernels()

    degt = sc_degree(dst)
    xw1 = _tc_matmul(x, W1)
    dis, y1 = _tc_prep(degt, xw1)
    acc1 = sc_scatter(y1, src, dst)
    y2 = _tc_layer(acc1, y1, dis, b1r, g1r, be1r, W2)
    acc2 = sc_scatter(y2, src, dst)
    return _tc_final(acc2, y2, dis, b2r, g2r, be2r, bf, Wl, blr)


# speedup vs baseline: 24.1794x; 1.0032x over previous
import functools
import jax
import jax.numpy as jnp
from jax import lax
from jax.experimental import pallas as pl
from jax.experimental.pallas import tpu as pltpu
from jax.experimental.pallas import tpu_sc as plsc

N = 10000
E = 320000
D = 128
H = 128
C = 40
G = 64

NC = 2
NS = 16
NW = NC * NS
EW = 128
NROWS = E // EW
CPW = NROWS // NW
NCH = CPW + 1
ZCH = 80
NZ = N // ZCH

RB = 2000
GRID = N // RB


@functools.cache
def _sc_kernels():
    mesh = plsc.VectorSubcoreMesh(core_axis_name="c", subcore_axis_name="s")

    deg_kernel = functools.partial(
        pl.kernel,
        out_type=jax.ShapeDtypeStruct((NC, N, H), jnp.float32),
        mesh=mesh,
        scratch_types=[
            pltpu.VMEM((NCH, 1, EW), jnp.int32),
            pltpu.VMEM((EW, H), jnp.float32),
            pltpu.VMEM((ZCH, H), jnp.float32),
            pltpu.VMEM_SHARED((N, H), jnp.float32),
            pltpu.SemaphoreType.DMA,
        ],
    )(_sc_degree_body)

    scat_kernel = functools.partial(
        pl.kernel,
        out_type=jax.ShapeDtypeStruct((NC, N, H), jnp.float32),
        mesh=mesh,
        scratch_types=[
            pltpu.VMEM((4 * EW,), jnp.int32),
            pltpu.VMEM((4 * EW,), jnp.int32),
            pltpu.VMEM((NCH, 1, EW), jnp.int32),
            pltpu.VMEM((EW, H), jnp.float32),
            pltpu.VMEM((EW, H), jnp.float32),
            pltpu.VMEM_SHARED((N, H), jnp.float32),
            pltpu.SemaphoreType.DMA,
            pltpu.SemaphoreType.DMA,
            pltpu.SemaphoreType.DMA,
            pltpu.SemaphoreType.DMA,
        ],
    )(_sc_scatter_body)

    return deg_kernel, scat_kernel


def _sc_degree_body(dst_hbm, out_hbm, didxb, ones_v, zbuf, dacc, sem0):
    cid = lax.axis_index("c")
    sid = lax.axis_index("s")
    wid = sid * NC + cid

    start = wid * CPW + jnp.minimum(wid, 4)
    ld = pltpu.async_copy(dst_hbm.at[pl.ds(start, CPW)],
                          didxb.at[pl.ds(0, CPW)], sem0)

    @pl.when(wid < 4)
    def _():
        pltpu.sync_copy(dst_hbm.at[start + CPW], didxb.at[CPW])

    @pl.loop(0, EW)
    def _(r):
        @pl.loop(0, H, step=16)
        def _(c):
            ones_v[r, pl.ds(c, 16)] = jnp.full((16,), 1.0, jnp.float32)

    @pl.loop(0, ZCH)
    def _(r):
        @pl.loop(0, H, step=16)
        def _(c):
            zbuf[r, pl.ds(c, 16)] = jnp.zeros((16,), jnp.float32)

    @pl.loop(sid, NZ, step=NS)
    def _(k):
        pltpu.sync_copy(zbuf, dacc.at[pl.ds(k * ZCH, ZCH)])

    ld.wait()
    plsc.subcore_barrier()

    @pl.loop(0, CPW)
    def _(k):
        pltpu.sync_copy(ones_v, dacc.at[didxb.at[k, 0]], add=True)

    @pl.when(wid < 4)
    def _():
        pltpu.sync_copy(ones_v, dacc.at[didxb.at[CPW, 0]], add=True)

    plsc.subcore_barrier()

    @pl.loop(sid, NZ, step=NS)
    def _(k):
        pltpu.sync_copy(dacc.at[pl.ds(k * ZCH, ZCH)],
                        out_hbm.at[cid, pl.ds(k * ZCH, ZCH)])


def _sc_scatter_body(y_hbm, src_hbm, dst_hbm, out_hbm, sbigA, sbigB,
                     didxb, rows0, rows1, acc,
                     semA, semB, semg0, semg1):
    cid = lax.axis_index("c")
    sid = lax.axis_index("s")
    wid = sid * NC + cid

    start = wid * CPW + jnp.minimum(wid, 4)
    nch = CPW + jnp.where(wid < 4, 1, 0)

    ldd = pltpu.async_copy(dst_hbm.at[pl.ds(start, CPW)],
                           didxb.at[pl.ds(0, CPW)], semg0)

    @pl.when(wid < 4)
    def _():
        pltpu.sync_copy(dst_hbm.at[start + CPW], didxb.at[CPW])

    @pl.loop(0, ZCH)
    def _(r):
        @pl.loop(0, H, step=16)
        def _(col):
            rows0[r, pl.ds(col, 16)] = jnp.zeros((16,), jnp.float32)

    @pl.loop(sid, NZ, step=NS)
    def _(k):
        pltpu.sync_copy(rows0.at[pl.ds(0, ZCH)],
                        acc.at[pl.ds(k * ZCH, ZCH)])

    ldd.wait()
    plsc.subcore_barrier()

    def idx4_load(c0, sbuf, sem):
        return pltpu.async_copy(src_hbm.at[pl.ds((start + c0) * EW, 4 * EW)],
                                sbuf, sem)

    def idx4_wait(sbuf, sem):
        pltpu.make_async_copy(src_hbm.at[pl.ds(0, 4 * EW)], sbuf, sem).wait()

    def gather(sbuf, b, rows, sem):
        return pltpu.async_copy(y_hbm.at[sbuf.at[pl.ds(b * EW, EW)]],
                                rows, sem)

    def gather_dyn(sbuf, off, rows, sem):
        return pltpu.async_copy(y_hbm.at[sbuf.at[pl.ds(off * EW, EW)]],
                                rows, sem)

    def scatter_add(j, rows):
        pltpu.sync_copy(rows, acc.at[didxb.at[j, 0]], add=True)

    def process4(sbuf, j):
        g0 = gather(sbuf, 0, rows0, semg0)
        g1 = gather(sbuf, 1, rows1, semg1)
        g0.wait()
        scatter_add(j, rows0)
        g1.wait()
        g2 = gather(sbuf, 2, rows0, semg0)
        scatter_add(j + 1, rows1)
        g2.wait()
        g3 = gather(sbuf, 3, rows1, semg1)
        scatter_add(j + 2, rows0)
        g3.wait()
        scatter_add(j + 3, rows1)

    idx4_load(0, sbigA, semA)
    idx4_load(4, sbigB, semB)

    @pl.loop(0, CPW - 6, step=8)
    def _(j):
        idx4_wait(sbigA, semA)
        process4(sbigA, j)
        idx4_load(j + 8, sbigA, semA)
        idx4_wait(sbigB, semB)
        process4(sbigB, j + 4)
        idx4_load(jnp.minimum(j + 12, nch - 4), sbigB, semB)

    idx4_wait(sbigA, semA)
    process4(sbigA, CPW - 6)
    idx4_wait(sbigB, semB)
    off0 = (CPW - 2) - (nch - 4)
    tg0 = gather_dyn(sbigB, off0, rows0, semg0)
    tg1 = gather_dyn(sbigB, off0 + 1, rows1, semg1)
    tg0.wait()
    scatter_add(CPW - 2, rows0)
    tg1.wait()
    scatter_add(CPW - 1, rows1)

    @pl.when(wid < 4)
    def _():
        gather(sbigB, 3, rows0, semg0).wait()
        scatter_add(CPW, rows0)

    plsc.subcore_barrier()

    @pl.loop(sid, NZ, step=NS)
    def _(k):
        pltpu.sync_copy(acc.at[pl.ds(k * ZCH, ZCH)],
                        out_hbm.at[cid, pl.ds(k * ZCH, ZCH)])


def _mm_body(x_ref, w_ref, o_ref):
    o_ref[...] = jnp.dot(x_ref[...], w_ref[...],
                         preferred_element_type=jnp.float32)


def _tc_matmul(x, W):
    return pl.pallas_call(
        _mm_body,
        grid=(GRID,),
        in_specs=[pl.BlockSpec((RB, D), lambda i: (i, 0)),
                  pl.BlockSpec((D, H), lambda i: (0, 0))],
        out_specs=pl.BlockSpec((RB, H), lambda i: (i, 0)),
        out_shape=jax.ShapeDtypeStruct((N, H), jnp.float32),
    )(x, W)


def _prep_body(degt_ref, xw_ref, dis_ref, y_ref):
    deg = jnp.sum(degt_ref[0] + degt_ref[1], axis=-1, keepdims=True)
    deg = deg * (1.0 / H) + 1.0
    dis = lax.rsqrt(deg)
    dis_ref[...] = dis
    y_ref[...] = xw_ref[...] * dis


def _tc_prep(degt, xw):
    return pl.pallas_call(
        _prep_body,
        grid=(GRID,),
        in_specs=[pl.BlockSpec((NC, RB, H), lambda i: (0, i, 0)),
                  pl.BlockSpec((RB, H), lambda i: (i, 0))],
        out_specs=[pl.BlockSpec((RB, 1), lambda i: (i, 0)),
                   pl.BlockSpec((RB, H), lambda i: (i, 0))],
        out_shape=[jax.ShapeDtypeStruct((N, 1), jnp.float32),
                   jax.ShapeDtypeStruct((N, H), jnp.float32)],
    )(degt, xw)


def _ln_relu(t, g, be):
    m = jnp.mean(t, axis=-1, keepdims=True)
    cen = t - m
    v = jnp.mean(cen * cen, axis=-1, keepdims=True)
    h = cen * lax.rsqrt(v + 1e-5) * g + be
    return jnp.maximum(h, 0.0)


def _layer_body(acc_ref, y_ref, dis_ref, b_ref, g_ref, be_ref, w2_ref, y2_ref):
    dis = dis_ref[...]
    t = (acc_ref[0] + acc_ref[1] + y_ref[...]) * dis + b_ref[...]
    h = _ln_relu(t, g_ref[...], be_ref[...])
    y2_ref[...] = jnp.dot(h, w2_ref[...],
                          preferred_element_type=jnp.float32) * dis


def _tc_layer(acc, y, dis, b, g, be, W2):
    return pl.pallas_call(
        _layer_body,
        grid=(GRID,),
        in_specs=[pl.BlockSpec((NC, RB, H), lambda i: (0, i, 0)),
                  pl.BlockSpec((RB, H), lambda i: (i, 0)),
                  pl.BlockSpec((RB, 1), lambda i: (i, 0)),
                  pl.BlockSpec((1, H), lambda i: (0, 0)),
                  pl.BlockSpec((1, H), lambda i: (0, 0)),
                  pl.BlockSpec((1, H), lambda i: (0, 0)),
                  pl.BlockSpec((H, H), lambda i: (0, 0))],
        out_specs=pl.BlockSpec((RB, H), lambda i: (i, 0)),
        out_shape=jax.ShapeDtypeStruct((N, H), jnp.float32),
    )(acc, y, dis, b, g, be, W2)


def _final_body(acc_ref, y_ref, dis_ref, b_ref, g_ref, be_ref, bf_ref,
                wl_ref, bl_ref, o_ref, psum, pcnt):
    i = pl.program_id(0)

    @pl.when(i == 0)
    def _():
        psum[...] = jnp.zeros_like(psum)
        pcnt[...] = jnp.zeros_like(pcnt)

    dis = dis_ref[...]
    t = (acc_ref[0] + acc_ref[1] + y_ref[...]) * dis + b_ref[...]
    h = _ln_relu(t, g_ref[...], be_ref[...])

    gids = lax.broadcasted_iota(jnp.int32, (RB, G), 1)
    onehot = (bf_ref[...] == gids).astype(jnp.float32)
    psum[...] += lax.dot_general(onehot, h, (((0,), (0,)), ((), ())),
                                 preferred_element_type=jnp.float32)
    pcnt[...] += lax.dot_general(onehot, jnp.ones((RB, H), jnp.float32),
                                 (((0,), (0,)), ((), ())),
                                 preferred_element_type=jnp.float32)

    @pl.when(i == GRID - 1)
    def _():
        pooled = psum[...] / jnp.maximum(pcnt[...], 1.0)
        o_ref[...] = jnp.dot(pooled, wl_ref[...],
                             preferred_element_type=jnp.float32) + bl_ref[...]


def _tc_final(acc, y, dis, b, g, be, bf, Wl, bl):
    return pl.pallas_call(
        _final_body,
        grid=(GRID,),
        in_specs=[pl.BlockSpec((NC, RB, H), lambda i: (0, i, 0)),
                  pl.BlockSpec((RB, H), lambda i: (i, 0)),
                  pl.BlockSpec((RB, 1), lambda i: (i, 0)),
                  pl.BlockSpec((1, H), lambda i: (0, 0)),
                  pl.BlockSpec((1, H), lambda i: (0, 0)),
                  pl.BlockSpec((1, H), lambda i: (0, 0)),
                  pl.BlockSpec((RB, 1), lambda i: (i, 0)),
                  pl.BlockSpec((H, C), lambda i: (0, 0)),
                  pl.BlockSpec((1, C), lambda i: (0, 0))],
        out_specs=pl.BlockSpec((G, C), lambda i: (0, 0)),
        out_shape=jax.ShapeDtypeStruct((G, C), jnp.float32),
        scratch_shapes=[pltpu.VMEM((G, H), jnp.float32),
                        pltpu.VMEM((G, H), jnp.float32)],
    )(acc, y, dis, b, g, be, bf, Wl, bl)


def kernel(x, edge_index, batch, W1, b1, g1, be1, W2, b2, g2, be2, Wl, bl):
    src = edge_index[0]
    dst = edge_index[1].reshape(NROWS, 1, EW)
    bf = batch.reshape(N, 1)
    b1r, g1r, be1r = b1.reshape(1, H), g1.reshape(1, H), be1.reshape(1, H)
    b2r, g2r, be2r = b2.reshape(1, H), g2.reshape(1, H), be2.reshape(1, H)
    blr = bl.reshape(1, C)

    sc_degree, sc_scatter = _sc_kernels()

    degt = sc_degree(dst)
    xw1 = _tc_matmul(x, W1)
    dis, y1 = _tc_prep(degt, xw1)
    acc1 = sc_scatter(y1, src, dst)
    y2 = _tc_layer(acc1, y1, dis, b1r, g1r, be1r, W2)
    acc2 = sc_scatter(y2, src, dst)
    return _tc_final(acc2, y2, dis, b2r, g2r, be2r, bf, Wl, blr)


# speedup vs baseline: 24.5421x; 1.0150x over previous
import functools
import jax
import jax.numpy as jnp
from jax import lax
from jax.experimental import pallas as pl
from jax.experimental.pallas import tpu as pltpu
from jax.experimental.pallas import tpu_sc as plsc

N = 10000
E = 320000
D = 128
H = 128
C = 40
G = 64

NC = 2
NS = 16
NW = NC * NS
EW = 128
NROWS = E // EW
CPW = NROWS // NW
NCH = CPW + 1
ZCH = 80
NZ = N // ZCH

RB = 2000
GRID = N // RB


@functools.cache
def _sc_kernels():
    mesh = plsc.VectorSubcoreMesh(core_axis_name="c", subcore_axis_name="s")

    deg_kernel = functools.partial(
        pl.kernel,
        out_type=jax.ShapeDtypeStruct((NC, N, H), jnp.float32),
        mesh=mesh,
        scratch_types=[
            pltpu.VMEM((NCH, 1, EW), jnp.int32),
            pltpu.VMEM((EW, H), jnp.float32),
            pltpu.VMEM((ZCH, H), jnp.float32),
            pltpu.VMEM_SHARED((N, H), jnp.float32),
            pltpu.SemaphoreType.DMA,
        ],
    )(_sc_degree_body)

    scat_kernel = functools.partial(
        pl.kernel,
        out_type=jax.ShapeDtypeStruct((NC, N, H), jnp.float32),
        mesh=mesh,
        scratch_types=[
            pltpu.VMEM((4 * EW,), jnp.int32),
            pltpu.VMEM((4 * EW,), jnp.int32),
            pltpu.VMEM((NCH, 1, EW), jnp.int32),
            pltpu.VMEM((EW, H), jnp.float32),
            pltpu.VMEM((EW, H), jnp.float32),
            pltpu.VMEM_SHARED((N, H), jnp.float32),
            pltpu.SemaphoreType.DMA,
            pltpu.SemaphoreType.DMA,
            pltpu.SemaphoreType.DMA,
            pltpu.SemaphoreType.DMA,
            pltpu.SemaphoreType.DMA,
            pltpu.SemaphoreType.DMA,
        ],
    )(_sc_scatter_body)

    return deg_kernel, scat_kernel


def _sc_degree_body(dst_hbm, out_hbm, didxb, ones_v, zbuf, dacc, sem0):
    cid = lax.axis_index("c")
    sid = lax.axis_index("s")
    wid = sid * NC + cid

    start = wid * CPW + jnp.minimum(wid, 4)
    ld = pltpu.async_copy(dst_hbm.at[pl.ds(start, CPW)],
                          didxb.at[pl.ds(0, CPW)], sem0)

    @pl.when(wid < 4)
    def _():
        pltpu.sync_copy(dst_hbm.at[start + CPW], didxb.at[CPW])

    @pl.loop(0, EW)
    def _(r):
        @pl.loop(0, H, step=16)
        def _(c):
            ones_v[r, pl.ds(c, 16)] = jnp.full((16,), 1.0, jnp.float32)

    @pl.loop(0, ZCH)
    def _(r):
        @pl.loop(0, H, step=16)
        def _(c):
            zbuf[r, pl.ds(c, 16)] = jnp.zeros((16,), jnp.float32)

    @pl.loop(sid, NZ, step=NS)
    def _(k):
        pltpu.sync_copy(zbuf, dacc.at[pl.ds(k * ZCH, ZCH)])

    ld.wait()
    plsc.subcore_barrier()

    @pl.loop(0, CPW)
    def _(k):
        pltpu.sync_copy(ones_v, dacc.at[didxb.at[k, 0]], add=True)

    @pl.when(wid < 4)
    def _():
        pltpu.sync_copy(ones_v, dacc.at[didxb.at[CPW, 0]], add=True)

    plsc.subcore_barrier()

    @pl.loop(sid, NZ, step=NS)
    def _(k):
        pltpu.sync_copy(dacc.at[pl.ds(k * ZCH, ZCH)],
                        out_hbm.at[cid, pl.ds(k * ZCH, ZCH)])


def _sc_scatter_body(y_hbm, src_hbm, dst_hbm, out_hbm, sbigA, sbigB,
                     didxb, rows0, rows1, acc,
                     semA, semB, semg0, semg1, semh0, semh1):
    cid = lax.axis_index("c")
    sid = lax.axis_index("s")
    wid = sid * NC + cid

    start = wid * CPW + jnp.minimum(wid, 4)
    nch = CPW + jnp.where(wid < 4, 1, 0)

    ldd = pltpu.async_copy(dst_hbm.at[pl.ds(start, CPW)],
                           didxb.at[pl.ds(0, CPW)], semg0)

    @pl.when(wid < 4)
    def _():
        pltpu.sync_copy(dst_hbm.at[start + CPW], didxb.at[CPW])

    @pl.loop(0, ZCH)
    def _(r):
        @pl.loop(0, H, step=16)
        def _(col):
            rows0[r, pl.ds(col, 16)] = jnp.zeros((16,), jnp.float32)

    @pl.loop(sid, NZ, step=NS)
    def _(k):
        pltpu.sync_copy(rows0.at[pl.ds(0, ZCH)],
                        acc.at[pl.ds(k * ZCH, ZCH)])

    ldd.wait()
    plsc.subcore_barrier()

    def idx4_load(c0, sbuf, sem):
        return pltpu.async_copy(src_hbm.at[pl.ds((start + c0) * EW, 4 * EW)],
                                sbuf, sem)

    def idx4_wait(sbuf, sem):
        pltpu.make_async_copy(src_hbm.at[pl.ds(0, 4 * EW)], sbuf, sem).wait()

    HEW = EW // 2

    def gather(sbuf, b, rows, sem, semh):
        ga = pltpu.async_copy(y_hbm.at[sbuf.at[pl.ds(b * EW, HEW)]],
                              rows.at[pl.ds(0, HEW)], sem)
        gb = pltpu.async_copy(y_hbm.at[sbuf.at[pl.ds(b * EW + HEW, HEW)]],
                              rows.at[pl.ds(HEW, HEW)], semh)
        return ga, gb

    def gather_dyn(sbuf, off, rows, sem):
        return pltpu.async_copy(y_hbm.at[sbuf.at[pl.ds(off * EW, EW)]],
                                rows, sem)

    def scatter_add(j, rows):
        pltpu.sync_copy(rows, acc.at[didxb.at[j, 0]], add=True)

    def process4(sbuf, j):
        g0a, g0b = gather(sbuf, 0, rows0, semg0, semh0)
        g1a, g1b = gather(sbuf, 1, rows1, semg1, semh1)
        g0a.wait()
        g0b.wait()
        scatter_add(j, rows0)
        g1a.wait()
        g1b.wait()
        g2a, g2b = gather(sbuf, 2, rows0, semg0, semh0)
        scatter_add(j + 1, rows1)
        g2a.wait()
        g2b.wait()
        g3a, g3b = gather(sbuf, 3, rows1, semg1, semh1)
        scatter_add(j + 2, rows0)
        g3a.wait()
        g3b.wait()
        scatter_add(j + 3, rows1)

    idx4_load(0, sbigA, semA)
    idx4_load(4, sbigB, semB)

    @pl.loop(0, CPW - 6, step=8)
    def _(j):
        idx4_wait(sbigA, semA)
        process4(sbigA, j)
        idx4_load(j + 8, sbigA, semA)
        idx4_wait(sbigB, semB)
        process4(sbigB, j + 4)
        idx4_load(jnp.minimum(j + 12, nch - 4), sbigB, semB)

    idx4_wait(sbigA, semA)
    process4(sbigA, CPW - 6)
    idx4_wait(sbigB, semB)
    off0 = (CPW - 2) - (nch - 4)
    tg0 = gather_dyn(sbigB, off0, rows0, semg0)
    tg1 = gather_dyn(sbigB, off0 + 1, rows1, semg1)
    tg0.wait()
    scatter_add(CPW - 2, rows0)
    tg1.wait()
    scatter_add(CPW - 1, rows1)

    @pl.when(wid < 4)
    def _():
        gather_dyn(sbigB, 3, rows0, semg0).wait()
        scatter_add(CPW, rows0)

    plsc.subcore_barrier()

    @pl.loop(sid, NZ, step=NS)
    def _(k):
        pltpu.sync_copy(acc.at[pl.ds(k * ZCH, ZCH)],
                        out_hbm.at[cid, pl.ds(k * ZCH, ZCH)])


def _mm_body(x_ref, w_ref, o_ref):
    o_ref[...] = jnp.dot(x_ref[...], w_ref[...],
                         preferred_element_type=jnp.float32)


def _tc_matmul(x, W):
    return pl.pallas_call(
        _mm_body,
        grid=(GRID,),
        in_specs=[pl.BlockSpec((RB, D), lambda i: (i, 0)),
                  pl.BlockSpec((D, H), lambda i: (0, 0))],
        out_specs=pl.BlockSpec((RB, H), lambda i: (i, 0)),
        out_shape=jax.ShapeDtypeStruct((N, H), jnp.float32),
    )(x, W)


def _prep_body(degt_ref, xw_ref, dis_ref, y_ref):
    deg = jnp.sum(degt_ref[0] + degt_ref[1], axis=-1, keepdims=True)
    deg = deg * (1.0 / H) + 1.0
    dis = lax.rsqrt(deg)
    dis_ref[...] = dis
    y_ref[...] = xw_ref[...] * dis


def _tc_prep(degt, xw):
    return pl.pallas_call(
        _prep_body,
        grid=(GRID,),
        in_specs=[pl.BlockSpec((NC, RB, H), lambda i: (0, i, 0)),
                  pl.BlockSpec((RB, H), lambda i: (i, 0))],
        out_specs=[pl.BlockSpec((RB, 1), lambda i: (i, 0)),
                   pl.BlockSpec((RB, H), lambda i: (i, 0))],
        out_shape=[jax.ShapeDtypeStruct((N, 1), jnp.float32),
                   jax.ShapeDtypeStruct((N, H), jnp.float32)],
    )(degt, xw)


def _ln_relu(t, g, be):
    m = jnp.mean(t, axis=-1, keepdims=True)
    cen = t - m
    v = jnp.mean(cen * cen, axis=-1, keepdims=True)
    h = cen * lax.rsqrt(v + 1e-5) * g + be
    return jnp.maximum(h, 0.0)


def _layer_body(acc_ref, y_ref, dis_ref, b_ref, g_ref, be_ref, w2_ref, y2_ref):
    dis = dis_ref[...]
    t = (acc_ref[0] + acc_ref[1] + y_ref[...]) * dis + b_ref[...]
    h = _ln_relu(t, g_ref[...], be_ref[...])
    y2_ref[...] = jnp.dot(h, w2_ref[...],
                          preferred_element_type=jnp.float32) * dis


def _tc_layer(acc, y, dis, b, g, be, W2):
    return pl.pallas_call(
        _layer_body,
        grid=(GRID,),
        in_specs=[pl.BlockSpec((NC, RB, H), lambda i: (0, i, 0)),
                  pl.BlockSpec((RB, H), lambda i: (i, 0)),
                  pl.BlockSpec((RB, 1), lambda i: (i, 0)),
                  pl.BlockSpec((1, H), lambda i: (0, 0)),
                  pl.BlockSpec((1, H), lambda i: (0, 0)),
                  pl.BlockSpec((1, H), lambda i: (0, 0)),
                  pl.BlockSpec((H, H), lambda i: (0, 0))],
        out_specs=pl.BlockSpec((RB, H), lambda i: (i, 0)),
        out_shape=jax.ShapeDtypeStruct((N, H), jnp.float32),
    )(acc, y, dis, b, g, be, W2)


def _final_body(acc_ref, y_ref, dis_ref, b_ref, g_ref, be_ref, bf_ref,
                wl_ref, bl_ref, o_ref, psum, pcnt):
    i = pl.program_id(0)

    @pl.when(i == 0)
    def _():
        psum[...] = jnp.zeros_like(psum)
        pcnt[...] = jnp.zeros_like(pcnt)

    dis = dis_ref[...]
    t = (acc_ref[0] + acc_ref[1] + y_ref[...]) * dis + b_ref[...]
    h = _ln_relu(t, g_ref[...], be_ref[...])

    gids = lax.broadcasted_iota(jnp.int32, (RB, G), 1)
    onehot = (bf_ref[...] == gids).astype(jnp.float32)
    psum[...] += lax.dot_general(onehot, h, (((0,), (0,)), ((), ())),
                                 preferred_element_type=jnp.float32)
    pcnt[...] += lax.dot_general(onehot, jnp.ones((RB, H), jnp.float32),
                                 (((0,), (0,)), ((), ())),
                                 preferred_element_type=jnp.float32)

    @pl.when(i == GRID - 1)
    def _():
        pooled = psum[...] / jnp.maximum(pcnt[...], 1.0)
        o_ref[...] = jnp.dot(pooled, wl_ref[...],
                             preferred_element_type=jnp.float32) + bl_ref[...]


def _tc_final(acc, y, dis, b, g, be, bf, Wl, bl):
    return pl.pallas_call(
        _final_body,
        grid=(GRID,),
        in_specs=[pl.BlockSpec((NC, RB, H), lambda i: (0, i, 0)),
                  pl.BlockSpec((RB, H), lambda i: (i, 0)),
                  pl.BlockSpec((RB, 1), lambda i: (i, 0)),
                  pl.BlockSpec((1, H), lambda i: (0, 0)),
                  pl.BlockSpec((1, H), lambda i: (0, 0)),
                  pl.BlockSpec((1, H), lambda i: (0, 0)),
                  pl.BlockSpec((RB, 1), lambda i: (i, 0)),
                  pl.BlockSpec((H, C), lambda i: (0, 0)),
                  pl.BlockSpec((1, C), lambda i: (0, 0))],
        out_specs=pl.BlockSpec((G, C), lambda i: (0, 0)),
        out_shape=jax.ShapeDtypeStruct((G, C), jnp.float32),
        scratch_shapes=[pltpu.VMEM((G, H), jnp.float32),
                        pltpu.VMEM((G, H), jnp.float32)],
    )(acc, y, dis, b, g, be, bf, Wl, bl)


def kernel(x, edge_index, batch, W1, b1, g1, be1, W2, b2, g2, be2, Wl, bl):
    src = edge_index[0]
    dst = edge_index[1].reshape(NROWS, 1, EW)
    bf = batch.reshape(N, 1)
    b1r, g1r, be1r = b1.reshape(1, H), g1.reshape(1, H), be1.reshape(1, H)
    b2r, g2r, be2r = b2.reshape(1, H), g2.reshape(1, H), be2.reshape(1, H)
    blr = bl.reshape(1, C)

    sc_degree, sc_scatter = _sc_kernels()

    degt = sc_degree(dst)
    xw1 = _tc_matmul(x, W1)
    dis, y1 = _tc_prep(degt, xw1)
    acc1 = sc_scatter(y1, src, dst)
    y2 = _tc_layer(acc1, y1, dis, b1r, g1r, be1r, W2)
    acc2 = sc_scatter(y2, src, dst)
    return _tc_final(acc2, y2, dis, b2r, g2r, be2r, bf, Wl, blr)
